# baseline (speedup 1.0000x reference)
import jax
import jax.numpy as jnp
from jax.experimental import pallas as pl
from jax.experimental.pallas import tpu as pltpu

_B, _N, _C, _H, _W = 2, 512, 256, 7, 7
_D_IN = _C * _H * _W
_D_HID = 1024
_NCLS = 11
_NREG = 5
_OUT = _NCLS + _NREG
_M = _B * _N

_TM = 1024
_TK = 1792


def _mlp_kernel(x_ref, w1_ref, b1_ref, w2_ref, b2_ref, wc_ref, bc_ref,
                wr_ref, br_ref, o_ref, acc_ref):
    k = pl.program_id(1)

    @pl.when(k == 0)
    def _init():
        acc_ref[...] = jnp.zeros_like(acc_ref)

    acc_ref[...] += jnp.dot(x_ref[...].astype(jnp.bfloat16),
                            w1_ref[...].astype(jnp.bfloat16),
                            preferred_element_type=jnp.float32)

    @pl.when(k == pl.num_programs(1) - 1)
    def _finish():
        h1 = jnp.maximum(acc_ref[...] + b1_ref[...][None, :], 0.0)
        h2 = jnp.maximum(
            jnp.dot(h1.astype(jnp.bfloat16),
                    w2_ref[...].astype(jnp.bfloat16),
                    preferred_element_type=jnp.float32)
            + b2_ref[...][None, :], 0.0)
        cls = (jnp.dot(h2, wc_ref[...], preferred_element_type=jnp.float32)
               + bc_ref[...][None, :])
        reg = (jnp.dot(h2, wr_ref[...], preferred_element_type=jnp.float32)
               + br_ref[...][None, :])
        o_ref[:, :_NCLS] = cls
        o_ref[:, _NCLS:] = reg


def kernel(aligned_feat, W1, b1, W2, b2, Wc, bc, Wr, br):
    x = aligned_feat.reshape(_M, _D_IN)

    grid = (_M // _TM, _D_IN // _TK)
    out = pl.pallas_call(
        _mlp_kernel,
        grid=grid,
        in_specs=[
            pl.BlockSpec((_TM, _TK), lambda m, k: (m, k)),
            pl.BlockSpec((_TK, _D_HID), lambda m, k: (k, 0)),
            pl.BlockSpec((_D_HID,), lambda m, k: (0,)),
            pl.BlockSpec((_D_HID, _D_HID), lambda m, k: (0, 0)),
            pl.BlockSpec((_D_HID,), lambda m, k: (0,)),
            pl.BlockSpec((_D_HID, _NCLS), lambda m, k: (0, 0)),
            pl.BlockSpec((_NCLS,), lambda m, k: (0,)),
            pl.BlockSpec((_D_HID, _NREG), lambda m, k: (0, 0)),
            pl.BlockSpec((_NREG,), lambda m, k: (0,)),
        ],
        out_specs=pl.BlockSpec((_TM, _OUT), lambda m, k: (m, 0)),
        out_shape=jax.ShapeDtypeStruct((_M, _OUT), jnp.float32),
        scratch_shapes=[pltpu.VMEM((_TM, _D_HID), jnp.float32)],
        compiler_params=pltpu.CompilerParams(
            dimension_semantics=("parallel", "arbitrary")),
    )(x, W1, b1, W2, b2, Wc, bc, Wr, br)
    return out.reshape(_B, _N, _OUT)


# speedup vs baseline: 1.0017x; 1.0017x over previous
import jax
import jax.numpy as jnp
from jax.experimental import pallas as pl
from jax.experimental.pallas import tpu as pltpu

_B, _N, _C, _H, _W = 2, 512, 256, 7, 7
_D_IN = _C * _H * _W
_D_HID = 1024
_NCLS = 11
_NREG = 5
_OUT = _NCLS + _NREG
_M = _B * _N

_TM = 1024
_TK = 1792


def _mlp_kernel(x_ref, w1_ref, b1_ref, w2_ref, b2_ref, wc_ref, bc_ref,
                wr_ref, br_ref, o_ref, acc_ref):
    k = pl.program_id(1)

    @pl.when(k == 0)
    def _init():
        acc_ref[...] = jnp.zeros_like(acc_ref)

    acc_ref[...] += jnp.dot(x_ref[...], w1_ref[...],
                            preferred_element_type=jnp.float32)

    @pl.when(k == pl.num_programs(1) - 1)
    def _finish():
        h1 = jnp.maximum(acc_ref[...] + b1_ref[...][None, :], 0.0)
        h2 = jnp.maximum(
            jnp.dot(h1, w2_ref[...], preferred_element_type=jnp.float32)
            + b2_ref[...][None, :], 0.0)
        cls = (jnp.dot(h2, wc_ref[...], preferred_element_type=jnp.float32)
               + bc_ref[...][None, :])
        reg = (jnp.dot(h2, wr_ref[...], preferred_element_type=jnp.float32)
               + br_ref[...][None, :])
        o_ref[:, :_NCLS] = cls
        o_ref[:, _NCLS:] = reg


def kernel(aligned_feat, W1, b1, W2, b2, Wc, bc, Wr, br):
    x = aligned_feat.reshape(_M, _D_IN)

    grid = (_M // _TM, _D_IN // _TK)
    out = pl.pallas_call(
        _mlp_kernel,
        grid=grid,
        in_specs=[
            pl.BlockSpec((_TM, _TK), lambda m, k: (m, k)),
            pl.BlockSpec((_TK, _D_HID), lambda m, k: (k, 0)),
            pl.BlockSpec((_D_HID,), lambda m, k: (0,)),
            pl.BlockSpec((_D_HID, _D_HID), lambda m, k: (0, 0)),
            pl.BlockSpec((_D_HID,), lambda m, k: (0,)),
            pl.BlockSpec((_D_HID, _NCLS), lambda m, k: (0, 0)),
            pl.BlockSpec((_NCLS,), lambda m, k: (0,)),
            pl.BlockSpec((_D_HID, _NREG), lambda m, k: (0, 0)),
            pl.BlockSpec((_NREG,), lambda m, k: (0,)),
        ],
        out_specs=pl.BlockSpec((_TM, _OUT), lambda m, k: (m, 0)),
        out_shape=jax.ShapeDtypeStruct((_M, _OUT), jnp.float32),
        scratch_shapes=[pltpu.VMEM((_TM, _D_HID), jnp.float32)],
        compiler_params=pltpu.CompilerParams(
            dimension_semantics=("parallel", "arbitrary")),
    )(x, W1, b1, W2, b2, Wc, bc, Wr, br)
    return out.reshape(_B, _N, _OUT)
